# baseline (speedup 1.0000x reference)
import functools

import jax
import jax.numpy as jnp
import numpy as np
from jax import lax
from jax.experimental import pallas as pl
from jax.experimental.pallas import tpu as pltpu
from jax.experimental.pallas import tpu_sc as plsc

VOCAB = 100000
SEQ = 2048
BATCH = 4
HID = 768
EPS = 1e-12
N = BATCH * SEQ
NW = 32
SPW = SEQ // NW
CHS = 16
NCHUNK = SPW // CHS
NSEG = HID // 16
INV_H = 1.0 / HID


_GDN = lax.GatherDimensionNumbers(
    offset_dims=(), collapsed_slice_dims=(0,), start_index_map=(0,))


def _perm(x, idx):
    return lax.gather(x, idx, _GDN, slice_sizes=(1,),
                      mode=lax.GatherScatterMode.PROMISE_IN_BOUNDS)


def _lanesum(x):
    lane = lax.iota(jnp.int32, 16)
    for sh in (8, 4, 2, 1):
        x = x + _perm(x, (lane ^ sh).reshape(16, 1))
    return x


def _rsqrt16(x):
    xh = x * 0.5
    i = lax.bitcast_convert_type(x, jnp.int32)
    i = jnp.int32(0x5F3759DF) - lax.shift_right_logical(i, 1)
    y = lax.bitcast_convert_type(i, jnp.float32)
    for _ in range(3):
        y = y * (1.5 - xh * y * y)
    return y


def _make_fused():
    mesh = plsc.VectorSubcoreMesh(core_axis_name="c", subcore_axis_name="s")

    @functools.partial(
        pl.kernel,
        out_type=jax.ShapeDtypeStruct((N, HID), jnp.float32),
        mesh=mesh,
        scratch_types=[
            pltpu.VMEM((BATCH * SPW,), jnp.int32),
            pltpu.VMEM((BATCH * SPW,), jnp.float32),
            pltpu.VMEM((HID,), jnp.float32),
            pltpu.VMEM((HID,), jnp.float32),
            pltpu.VMEM((HID,), jnp.float32),
            pltpu.VMEM((HID,), jnp.float32),
            pltpu.VMEM((BATCH, CHS, HID), jnp.float32),
            pltpu.VMEM((BATCH, CHS, HID), jnp.float32),
            pltpu.VMEM((CHS, HID), jnp.float32),
            pltpu.VMEM((CHS, HID), jnp.float32),
            pltpu.SemaphoreType.DMA,
            pltpu.SemaphoreType.DMA,
            pltpu.SemaphoreType.DMA,
            pltpu.SemaphoreType.DMA,
        ],
    )
    def fused_k(ids_hbm, tts_hbm, table_hbm, pos_hbm, ttemb_hbm, gamma_hbm,
                beta_hbm, out_hbm, ids_v, tts_v, tt0_v, ttd_v, gamma_v,
                beta_v, buf_a, buf_b, pos_a, pos_b, sem_ga, sem_gb,
                sem_oa, sem_ob):
        wid = lax.axis_index("s") * 2 + lax.axis_index("c")
        s0 = wid * SPW

        for b in range(BATCH):
            pltpu.sync_copy(ids_hbm.at[pl.ds(b * SEQ + s0, SPW)],
                            ids_v.at[pl.ds(b * SPW, SPW)])
            pltpu.sync_copy(tts_hbm.at[pl.ds(b * SEQ + s0, SPW)],
                            tts_v.at[pl.ds(b * SPW, SPW)])

        def start_chunk(c, buf, posbuf, sem):
            for b in range(BATCH):
                pltpu.async_copy(
                    table_hbm.at[ids_v.at[pl.ds(b * SPW + c * CHS, CHS)]],
                    buf.at[b], sem)
            pltpu.async_copy(pos_hbm.at[pl.ds(s0 + c * CHS, CHS)],
                             posbuf, sem)

        def wait_chunk(buf, posbuf, sem):
            for b in range(BATCH):
                pltpu.make_async_copy(table_hbm.at[pl.ds(0, CHS)],
                                      buf.at[b], sem).wait()
            pltpu.make_async_copy(pos_hbm.at[pl.ds(0, CHS)],
                                  posbuf, sem).wait()

        start_chunk(0, buf_a, pos_a, sem_ga)
        start_chunk(1, buf_b, pos_b, sem_gb)

        pltpu.sync_copy(ttemb_hbm.at[0], tt0_v)
        pltpu.sync_copy(ttemb_hbm.at[1], ttd_v)
        pltpu.sync_copy(gamma_hbm, gamma_v)
        pltpu.sync_copy(beta_hbm, beta_v)
        for j in range(NSEG):
            sl = pl.ds(j * 16, 16)
            ttd_v[sl] = ttd_v[sl] - tt0_v[sl]

        def process(c, buf, posbuf):
            tt16 = [tts_v[pl.ds(b * SPW + c * CHS, CHS)]
                    for b in range(BATCH)]

            def body(s):
                idx = jnp.full((16, 1), s, jnp.int32)
                tt = [_perm(tt16[b], idx) for b in range(BATCH)]
                acc = [None] * BATCH
                acq = [None] * BATCH
                for j in range(NSEG):
                    sl = pl.ds(j * 16, 16)
                    pv = posbuf[s, sl] + tt0_v[sl]
                    tv = ttd_v[sl]
                    for b in range(BATCH):
                        v = buf[b, s, sl] + (pv + tt[b] * tv)
                        buf[b, s, sl] = v
                        if j == 0:
                            acc[b] = v
                            acq[b] = v * v
                        else:
                            acc[b] = acc[b] + v
                            acq[b] = acq[b] + v * v
                mv = [None] * BATCH
                rs = [None] * BATCH
                for b in range(BATCH):
                    m = _lanesum(acc[b]) * INV_H
                    q = _lanesum(acq[b]) * INV_H
                    var = q - m * m
                    mv[b] = m
                    rs[b] = _rsqrt16(var + EPS)
                for j in range(NSEG):
                    sl = pl.ds(j * 16, 16)
                    gv = gamma_v[sl]
                    bv = beta_v[sl]
                    for b in range(BATCH):
                        buf[b, s, sl] = ((buf[b, s, sl] - mv[b]) * rs[b]) \
                            * gv + bv
            plsc.parallel_loop(0, CHS)(body)

        def start_out(c, buf, sem):
            for b in range(BATCH):
                pltpu.async_copy(
                    buf.at[b],
                    out_hbm.at[pl.ds(b * SEQ + s0 + c * CHS, CHS)], sem)

        def wait_out(buf, sem):
            for b in range(BATCH):
                pltpu.make_async_copy(table_hbm.at[pl.ds(0, CHS)],
                                      buf.at[b], sem).wait()

        def chunk_pair(t, _):
            wait_chunk(buf_a, pos_a, sem_ga)
            process(2 * t, buf_a, pos_a)
            start_out(2 * t, buf_a, sem_oa)

            wait_chunk(buf_b, pos_b, sem_gb)
            process(2 * t + 1, buf_b, pos_b)
            start_out(2 * t + 1, buf_b, sem_ob)

            @pl.when(t < NCHUNK // 2 - 1)
            def _():
                wait_out(buf_a, sem_oa)
                start_chunk(2 * t + 2, buf_a, pos_a, sem_ga)
                wait_out(buf_b, sem_ob)
                start_chunk(2 * t + 3, buf_b, pos_b, sem_gb)
            return _
        lax.fori_loop(0, NCHUNK // 2, chunk_pair, None)
        wait_out(buf_a, sem_oa)
        wait_out(buf_b, sem_ob)

    return fused_k


_sc_fused = _make_fused()


def kernel(input_ids, token_type_ids, word_embeddings, token_type_embeddings,
           position_embeddings, ln_gamma, ln_beta):
    ids = input_ids.reshape(N).astype(jnp.int32)
    tts = token_type_ids.reshape(N).astype(jnp.float32)
    out = _sc_fused(ids, tts, word_embeddings, position_embeddings,
                    token_type_embeddings, ln_gamma, ln_beta)
    return out.reshape(BATCH, SEQ, HID)


# speedup vs baseline: 4.4554x; 4.4554x over previous
import functools

import jax
import jax.numpy as jnp
from jax import lax
from jax.experimental import pallas as pl
from jax.experimental.pallas import tpu as pltpu
from jax.experimental.pallas import tpu_sc as plsc

VOCAB = 100000
SEQ = 2048
BATCH = 4
HID = 768
EPS = 1e-12
N = BATCH * SEQ
NW = 32
TOK_PER_W = N // NW
CH = 64
NCH = TOK_PER_W // CH


def _make_sc_gather():
    mesh = plsc.VectorSubcoreMesh(core_axis_name="c", subcore_axis_name="s")

    @functools.partial(
        pl.kernel,
        out_type=jax.ShapeDtypeStruct((N, HID), jnp.float32),
        mesh=mesh,
        scratch_types=[
            pltpu.VMEM((TOK_PER_W,), jnp.int32),
            pltpu.VMEM((2, CH, HID), jnp.float32),
            pltpu.SemaphoreType.DMA,
            pltpu.SemaphoreType.DMA,
        ],
    )
    def gather_k(ids_hbm, table_hbm, out_hbm, idx_v, rows_v, sem0, sem1):
        wid = lax.axis_index("s") * 2 + lax.axis_index("c")
        base = wid * TOK_PER_W
        sems = (sem0, sem1)
        pltpu.sync_copy(ids_hbm.at[pl.ds(base, TOK_PER_W)], idx_v)
        copies = [
            pltpu.async_copy(
                table_hbm.at[idx_v.at[pl.ds(c * CH, CH)]],
                rows_v.at[c % 2], sems[c % 2])
            for c in range(1)
        ]
        for c in range(NCH):
            if c + 1 < NCH:
                copies.append(pltpu.async_copy(
                    table_hbm.at[idx_v.at[pl.ds((c + 1) * CH, CH)]],
                    rows_v.at[(c + 1) % 2], sems[(c + 1) % 2]))
            copies[c].wait()
            pltpu.sync_copy(rows_v.at[c % 2],
                            out_hbm.at[pl.ds(base + c * CH, CH)])

    return gather_k


_sc_gather = _make_sc_gather()

ROWS = 512
SBLK = SEQ // ROWS


def _ln_body(x_ref, pos_ref, tt_ref, ttemb_ref, gamma_ref, beta_ref, o_ref):
    x = x_ref[...] + pos_ref[...]
    ttf = tt_ref[...]
    ttv = ttemb_ref[0:1, :] + ttf * (ttemb_ref[1:2, :] - ttemb_ref[0:1, :])
    x = x + ttv
    mean = jnp.mean(x, axis=-1, keepdims=True)
    xc = x - mean
    var = jnp.mean(xc * xc, axis=-1, keepdims=True)
    xn = xc * lax.rsqrt(var + EPS)
    o_ref[...] = xn * gamma_ref[...][None, :] + beta_ref[...][None, :]


_ln_call = pl.pallas_call(
    _ln_body,
    grid=(SBLK, BATCH),
    in_specs=[
        pl.BlockSpec((ROWS, HID), lambda i, j: (j * SBLK + i, 0)),
        pl.BlockSpec((ROWS, HID), lambda i, j: (i, 0)),
        pl.BlockSpec((ROWS, 1), lambda i, j: (j * SBLK + i, 0)),
        pl.BlockSpec((8, HID), lambda i, j: (0, 0)),
        pl.BlockSpec((HID,), lambda i, j: (0,)),
        pl.BlockSpec((HID,), lambda i, j: (0,)),
    ],
    out_specs=pl.BlockSpec((ROWS, HID), lambda i, j: (j * SBLK + i, 0)),
    out_shape=jax.ShapeDtypeStruct((N, HID), jnp.float32),
)


def kernel(input_ids, token_type_ids, word_embeddings, token_type_embeddings,
           position_embeddings, ln_gamma, ln_beta):
    ids = input_ids.reshape(N).astype(jnp.int32)
    tts = token_type_ids.reshape(N, 1).astype(jnp.float32)
    ttemb = jnp.concatenate(
        [token_type_embeddings,
         jnp.zeros((6, HID), token_type_embeddings.dtype)], axis=0)
    x = _sc_gather(ids, word_embeddings)
    out = _ln_call(x, position_embeddings, tts, ttemb, ln_gamma, ln_beta)
    return out.reshape(BATCH, SEQ, HID)


# speedup vs baseline: 4.6907x; 1.0528x over previous
import functools

import jax
import jax.numpy as jnp
from jax import lax
from jax.experimental import pallas as pl
from jax.experimental.pallas import tpu as pltpu
from jax.experimental.pallas import tpu_sc as plsc

VOCAB = 100000
SEQ = 2048
BATCH = 4
HID = 768
EPS = 1e-12
N = BATCH * SEQ
NW = 32
TOK_PER_W = N // NW
CH = 64
NCH = TOK_PER_W // CH


def _make_sc_gather():
    mesh = plsc.VectorSubcoreMesh(core_axis_name="c", subcore_axis_name="s")

    @functools.partial(
        pl.kernel,
        out_type=jax.ShapeDtypeStruct((N, HID), jnp.float32),
        mesh=mesh,
        scratch_types=[
            pltpu.VMEM((TOK_PER_W,), jnp.int32),
            pltpu.VMEM((2, CH, HID), jnp.float32),
            pltpu.SemaphoreType.DMA,
            pltpu.SemaphoreType.DMA,
        ],
    )
    def gather_k(ids_hbm, table_hbm, out_hbm, idx_v, rows_v, sem0, sem1):
        wid = lax.axis_index("s") * 2 + lax.axis_index("c")
        base = wid * TOK_PER_W
        sems = (sem0, sem1)
        pltpu.sync_copy(ids_hbm.at[pl.ds(base, TOK_PER_W)], idx_v)
        copies = [
            pltpu.async_copy(
                table_hbm.at[idx_v.at[pl.ds(c * CH, CH)]],
                rows_v.at[c % 2], sems[c % 2])
            for c in range(1)
        ]
        for c in range(NCH):
            if c + 1 < NCH:
                copies.append(pltpu.async_copy(
                    table_hbm.at[idx_v.at[pl.ds((c + 1) * CH, CH)]],
                    rows_v.at[(c + 1) % 2], sems[(c + 1) % 2]))
            copies[c].wait()
            pltpu.sync_copy(rows_v.at[c % 2],
                            out_hbm.at[pl.ds(base + c * CH, CH)])

    return gather_k


_sc_gather = _make_sc_gather()

ROWS = 1024
SBLK = SEQ // ROWS


def _ln_body(x_ref, pos_ref, tt_ref, ttemb_ref, gamma_ref, beta_ref, o_ref):
    x = x_ref[...] + pos_ref[...]
    ttf = tt_ref[...]
    ttv = ttemb_ref[0:1, :] + ttf * (ttemb_ref[1:2, :] - ttemb_ref[0:1, :])
    x = x + ttv
    mean = jnp.mean(x, axis=-1, keepdims=True)
    xc = x - mean
    var = jnp.mean(xc * xc, axis=-1, keepdims=True)
    xn = xc * lax.rsqrt(var + EPS)
    o_ref[...] = xn * gamma_ref[...][None, :] + beta_ref[...][None, :]


_ln_call = pl.pallas_call(
    _ln_body,
    grid=(SBLK, BATCH),
    in_specs=[
        pl.BlockSpec((ROWS, HID), lambda i, j: (j * SBLK + i, 0)),
        pl.BlockSpec((ROWS, HID), lambda i, j: (i, 0)),
        pl.BlockSpec((ROWS, 1), lambda i, j: (j * SBLK + i, 0)),
        pl.BlockSpec((8, HID), lambda i, j: (0, 0)),
        pl.BlockSpec((HID,), lambda i, j: (0,)),
        pl.BlockSpec((HID,), lambda i, j: (0,)),
    ],
    out_specs=pl.BlockSpec((ROWS, HID), lambda i, j: (j * SBLK + i, 0)),
    out_shape=jax.ShapeDtypeStruct((N, HID), jnp.float32),
)


def kernel(input_ids, token_type_ids, word_embeddings, token_type_embeddings,
           position_embeddings, ln_gamma, ln_beta):
    ids = input_ids.reshape(N).astype(jnp.int32)
    tts = token_type_ids.reshape(N, 1).astype(jnp.float32)
    ttemb = jnp.concatenate(
        [token_type_embeddings,
         jnp.zeros((6, HID), token_type_embeddings.dtype)], axis=0)
    x = _sc_gather(ids, word_embeddings)
    out = _ln_call(x, position_embeddings, tts, ttemb, ln_gamma, ln_beta)
    return out.reshape(BATCH, SEQ, HID)


# speedup vs baseline: 4.7127x; 1.0047x over previous
import functools

import jax
import jax.numpy as jnp
from jax import lax
from jax.experimental import pallas as pl
from jax.experimental.pallas import tpu as pltpu
from jax.experimental.pallas import tpu_sc as plsc

VOCAB = 100000
SEQ = 2048
BATCH = 4
HID = 768
EPS = 1e-12
N = BATCH * SEQ
NW = 32
TOK_PER_W = N // NW
CH = 64
NCH = TOK_PER_W // CH


def _make_sc_gather():
    mesh = plsc.VectorSubcoreMesh(core_axis_name="c", subcore_axis_name="s")

    @functools.partial(
        pl.kernel,
        out_type=jax.ShapeDtypeStruct((N, HID), jnp.float32),
        mesh=mesh,
        scratch_types=[
            pltpu.VMEM((TOK_PER_W,), jnp.int32),
            pltpu.VMEM((2, CH, HID), jnp.float32),
            pltpu.SemaphoreType.DMA,
            pltpu.SemaphoreType.DMA,
        ],
    )
    def gather_k(ids_hbm, table_hbm, out_hbm, idx_v, rows_v, sem0, sem1):
        wid = lax.axis_index("s") * 2 + lax.axis_index("c")
        base = wid * TOK_PER_W
        sems = (sem0, sem1)
        pltpu.sync_copy(ids_hbm.at[pl.ds(base, TOK_PER_W)], idx_v)
        copies = [
            pltpu.async_copy(
                table_hbm.at[idx_v.at[pl.ds(c * CH, CH)]],
                rows_v.at[c % 2], sems[c % 2])
            for c in range(1)
        ]
        for c in range(NCH):
            if c + 1 < NCH:
                copies.append(pltpu.async_copy(
                    table_hbm.at[idx_v.at[pl.ds((c + 1) * CH, CH)]],
                    rows_v.at[(c + 1) % 2], sems[(c + 1) % 2]))
            copies[c].wait()
            pltpu.sync_copy(rows_v.at[c % 2],
                            out_hbm.at[pl.ds(base + c * CH, CH)])

    return gather_k


_sc_gather = _make_sc_gather()

ROWS = 2048
SBLK = SEQ // ROWS


def _ln_body(x_ref, pos_ref, tt_ref, ttemb_ref, gamma_ref, beta_ref, o_ref):
    x = x_ref[...] + pos_ref[...]
    ttf = tt_ref[...]
    ttv = ttemb_ref[0:1, :] + ttf * (ttemb_ref[1:2, :] - ttemb_ref[0:1, :])
    x = x + ttv
    mean = jnp.mean(x, axis=-1, keepdims=True)
    xc = x - mean
    var = jnp.mean(xc * xc, axis=-1, keepdims=True)
    xn = xc * lax.rsqrt(var + EPS)
    o_ref[...] = xn * gamma_ref[...][None, :] + beta_ref[...][None, :]


_ln_call = pl.pallas_call(
    _ln_body,
    grid=(SBLK, BATCH),
    in_specs=[
        pl.BlockSpec((ROWS, HID), lambda i, j: (j * SBLK + i, 0)),
        pl.BlockSpec((ROWS, HID), lambda i, j: (i, 0)),
        pl.BlockSpec((ROWS, 1), lambda i, j: (j * SBLK + i, 0)),
        pl.BlockSpec((8, HID), lambda i, j: (0, 0)),
        pl.BlockSpec((HID,), lambda i, j: (0,)),
        pl.BlockSpec((HID,), lambda i, j: (0,)),
    ],
    out_specs=pl.BlockSpec((ROWS, HID), lambda i, j: (j * SBLK + i, 0)),
    out_shape=jax.ShapeDtypeStruct((N, HID), jnp.float32),
)


def kernel(input_ids, token_type_ids, word_embeddings, token_type_embeddings,
           position_embeddings, ln_gamma, ln_beta):
    ids = input_ids.reshape(N).astype(jnp.int32)
    tts = token_type_ids.reshape(N, 1).astype(jnp.float32)
    ttemb = jnp.concatenate(
        [token_type_embeddings,
         jnp.zeros((6, HID), token_type_embeddings.dtype)], axis=0)
    x = _sc_gather(ids, word_embeddings)
    out = _ln_call(x, position_embeddings, tts, ttemb, ln_gamma, ln_beta)
    return out.reshape(BATCH, SEQ, HID)


# speedup vs baseline: 4.7482x; 1.0075x over previous
import functools

import jax
import jax.numpy as jnp
from jax import lax
from jax.experimental import pallas as pl
from jax.experimental.pallas import tpu as pltpu
from jax.experimental.pallas import tpu_sc as plsc

VOCAB = 100000
SEQ = 2048
BATCH = 4
HID = 768
EPS = 1e-12
N = BATCH * SEQ
NW = 32
TOK_PER_W = N // NW
CH = 32
NBUF = 4
NCH = TOK_PER_W // CH


def _make_sc_gather():
    mesh = plsc.VectorSubcoreMesh(core_axis_name="c", subcore_axis_name="s")

    @functools.partial(
        pl.kernel,
        out_type=jax.ShapeDtypeStruct((N, HID), jnp.float32),
        mesh=mesh,
        scratch_types=[
            pltpu.VMEM((TOK_PER_W,), jnp.int32),
            pltpu.VMEM((NBUF, CH, HID), jnp.float32),
        ] + [pltpu.SemaphoreType.DMA] * NBUF,
    )
    def gather_k(ids_hbm, table_hbm, out_hbm, idx_v, rows_v, *sems):
        wid = lax.axis_index("s") * 2 + lax.axis_index("c")
        base = wid * TOK_PER_W
        pltpu.sync_copy(ids_hbm.at[pl.ds(base, TOK_PER_W)], idx_v)

        def start(c):
            return pltpu.async_copy(
                table_hbm.at[idx_v.at[pl.ds(c * CH, CH)]],
                rows_v.at[c % NBUF], sems[c % NBUF])

        copies = [start(c) for c in range(NBUF - 1)]
        for c in range(NCH):
            if c + NBUF - 1 < NCH:
                copies.append(start(c + NBUF - 1))
            copies[c].wait()
            pltpu.sync_copy(rows_v.at[c % NBUF],
                            out_hbm.at[pl.ds(base + c * CH, CH)])

    return gather_k


_sc_gather = _make_sc_gather()

ROWS = 2048
SBLK = SEQ // ROWS


def _ln_body(x_ref, pos_ref, tt_ref, ttemb_ref, gamma_ref, beta_ref, o_ref):
    x = x_ref[...] + pos_ref[...]
    ttf = tt_ref[...]
    ttv = ttemb_ref[0:1, :] + ttf * (ttemb_ref[1:2, :] - ttemb_ref[0:1, :])
    x = x + ttv
    mean = jnp.mean(x, axis=-1, keepdims=True)
    xc = x - mean
    var = jnp.mean(xc * xc, axis=-1, keepdims=True)
    xn = xc * lax.rsqrt(var + EPS)
    o_ref[...] = xn * gamma_ref[...][None, :] + beta_ref[...][None, :]


_ln_call = pl.pallas_call(
    _ln_body,
    grid=(SBLK, BATCH),
    in_specs=[
        pl.BlockSpec((ROWS, HID), lambda i, j: (j * SBLK + i, 0)),
        pl.BlockSpec((ROWS, HID), lambda i, j: (i, 0)),
        pl.BlockSpec((ROWS, 1), lambda i, j: (j * SBLK + i, 0)),
        pl.BlockSpec((8, HID), lambda i, j: (0, 0)),
        pl.BlockSpec((HID,), lambda i, j: (0,)),
        pl.BlockSpec((HID,), lambda i, j: (0,)),
    ],
    out_specs=pl.BlockSpec((ROWS, HID), lambda i, j: (j * SBLK + i, 0)),
    out_shape=jax.ShapeDtypeStruct((N, HID), jnp.float32),
)


def kernel(input_ids, token_type_ids, word_embeddings, token_type_embeddings,
           position_embeddings, ln_gamma, ln_beta):
    ids = input_ids.reshape(N).astype(jnp.int32)
    tts = token_type_ids.reshape(N, 1).astype(jnp.float32)
    ttemb = jnp.concatenate(
        [token_type_embeddings,
         jnp.zeros((6, HID), token_type_embeddings.dtype)], axis=0)
    x = _sc_gather(ids, word_embeddings)
    out = _ln_call(x, position_embeddings, tts, ttemb, ln_gamma, ln_beta)
    return out.reshape(BATCH, SEQ, HID)
